# baseline (speedup 1.0000x reference)
import functools

import jax
import jax.numpy as jnp
from jax.experimental import pallas as pl

N = 10000
E = 320000
NODE = 128
EDGE = 16
TIME = 32
MEM = 128
HEADS = 4
HPD = 32
MSG = NODE + EDGE + TIME
L0_IN = NODE + MEM + TIME

BLK = 1000


def _dense1_body(nf_ref, nfd2_ref, ea_ref, te_ref, ten_ref, has_ref,
                 w1a_ref, w1c_ref, w1d_ref, w1e_ref, b1_ref, w2_ref, b2_ref,
                 wiht_ref, bih_ref, bhh_ref, w0_ref, a0_ref,
                 hh0_ref, s0_ref):
    nf = nf_ref[...]
    u = (nf @ w1a_ref[...] + nfd2_ref[...] @ w1c_ref[...]
         + ea_ref[...] @ w1d_ref[...] + te_ref[...] @ w1e_ref[...] + b1_ref[...])
    u = jnp.maximum(u, 0.0)
    msgs = u @ w2_ref[...] + b2_ref[...]
    gi = msgs @ wiht_ref[...] + bih_ref[...]
    gh = bhh_ref[...]
    ir, iz, inn = gi[:, :MEM], gi[:, MEM:2 * MEM], gi[:, 2 * MEM:]
    hr, hz, hn = gh[:, :MEM], gh[:, MEM:2 * MEM], gh[:, 2 * MEM:]
    r = jax.nn.sigmoid(ir + hr)
    z = jax.nn.sigmoid(iz + hz)
    nn_ = jnp.tanh(inn + r * hn)
    newh = (1.0 - z) * nn_
    memv = newh * has_ref[...]
    x = jnp.concatenate([nf, memv, ten_ref[...]], axis=1)
    hh0 = x @ w0_ref[...]
    hh0_ref[...] = hh0
    s0_ref[...] = hh0 @ a0_ref[...]


def _proj_body(x_ref, w_ref, a_ref, hh_ref, s_ref):
    hh = x_ref[...] @ w_ref[...]
    hh_ref[...] = hh
    s_ref[...] = hh @ a_ref[...]


def _cls_body(x_ref, w1_ref, b1_ref, w2_ref, b2_ref, out_ref):
    u = jnp.maximum(x_ref[...] @ w1_ref[...] + b1_ref[...], 0.0)
    out_ref[...] = u @ w2_ref[...] + b2_ref[...]


def _full(shape):
    return pl.BlockSpec(shape, lambda i: (0,) * len(shape))


def _rows(width):
    return pl.BlockSpec((BLK, width), lambda i: (i, 0))


def _gat_sparse(hh, s, src, dst, bflat):
    e = s[src, :HEADS] + s[dst, HEADS:]
    e = jnp.where(e >= 0, e, 0.2 * e)
    m = jax.ops.segment_max(e, dst, num_segments=N)
    m = jnp.where(jnp.isfinite(m), m, 0.0)
    ex = jnp.exp(e - m[dst])
    den = jax.ops.segment_sum(ex, dst, num_segments=N)
    w = jnp.repeat(ex, HPD, axis=1)
    acc = jax.ops.segment_sum(w * hh[src], dst, num_segments=N)
    return acc / (jnp.repeat(den, HPD, axis=1) + 1e-16) + bflat


def kernel(node_features, edge_index, edge_attr, edge_times, time_w, time_b,
           msg_W1, msg_b1, msg_W2, msg_b2, gru_Wih, gru_Whh, gru_bih, gru_bhh,
           gat0_W, gat0_asrc, gat0_adst, gat0_b,
           gat1_W, gat1_asrc, gat1_adst, gat1_b,
           cls_W1, cls_b1, cls_W2, cls_b2):
    src = edge_index[0]
    dst = edge_index[1]
    eid = jnp.arange(E, dtype=jnp.int32)
    last_e = jnp.full((N,), -1, jnp.int32).at[src].max(eid)
    has = last_e >= 0
    le = jnp.maximum(last_e, 0)
    d2 = dst[le]
    nfd2 = node_features[d2]
    ea = edge_attr[le]
    t = jnp.where(has, edge_times[le], 0.0)
    te = jnp.sin(time_w * t[:, None] + time_b)
    has_f = has.astype(jnp.float32)[:, None]

    w1a = msg_W1[:NODE]
    w1c = msg_W1[NODE + MEM:NODE + MEM + NODE]
    w1d = msg_W1[NODE + MEM + NODE:NODE + MEM + NODE + EDGE]
    w1e = msg_W1[NODE + MEM + NODE + EDGE:]
    wiht = gru_Wih.T
    w0cat = jnp.transpose(gat0_W, (1, 0, 2)).reshape(L0_IN, HEADS * HPD)
    w1cat = jnp.transpose(gat1_W, (1, 0, 2)).reshape(NODE, HEADS * HPD)

    def amat(asrc, adst):
        a = jnp.zeros((HEADS * HPD, 2 * HEADS), jnp.float32)
        for h in range(HEADS):
            a = a.at[h * HPD:(h + 1) * HPD, h].set(asrc[h])
            a = a.at[h * HPD:(h + 1) * HPD, HEADS + h].set(adst[h])
        return a

    a0 = amat(gat0_asrc, gat0_adst)
    a1 = amat(gat1_asrc, gat1_adst)
    b0flat = gat0_b.reshape(-1)
    b1flat = gat1_b.reshape(-1)

    grid = (N // BLK,)
    hh0, s0 = pl.pallas_call(
        _dense1_body,
        grid=grid,
        in_specs=[
            _rows(NODE), _rows(NODE), _rows(EDGE), _rows(TIME), _rows(TIME),
            _rows(1),
            _full((NODE, MSG)), _full((NODE, MSG)), _full((EDGE, MSG)),
            _full((TIME, MSG)), _full((1, MSG)), _full((MSG, MSG)),
            _full((1, MSG)), _full((MSG, 3 * MEM)), _full((1, 3 * MEM)),
            _full((1, 3 * MEM)), _full((L0_IN, HEADS * HPD)),
            _full((HEADS * HPD, 2 * HEADS)),
        ],
        out_specs=[_rows(HEADS * HPD), _rows(2 * HEADS)],
        out_shape=[
            jax.ShapeDtypeStruct((N, HEADS * HPD), jnp.float32),
            jax.ShapeDtypeStruct((N, 2 * HEADS), jnp.float32),
        ],
    )(node_features, nfd2, ea, te, te,
      has_f, w1a, w1c, w1d, w1e, msg_b1[None, :], msg_W2, msg_b2[None, :],
      wiht, gru_bih[None, :], gru_bhh[None, :], w0cat, a0)

    x1 = _gat_sparse(hh0, s0, src, dst, b0flat)

    hh1, s1 = pl.pallas_call(
        _proj_body,
        grid=grid,
        in_specs=[_rows(NODE), _full((NODE, HEADS * HPD)),
                  _full((HEADS * HPD, 2 * HEADS))],
        out_specs=[_rows(HEADS * HPD), _rows(2 * HEADS)],
        out_shape=[
            jax.ShapeDtypeStruct((N, HEADS * HPD), jnp.float32),
            jax.ShapeDtypeStruct((N, 2 * HEADS), jnp.float32),
        ],
    )(x1, w1cat, a1)

    x2 = _gat_sparse(hh1, s1, src, dst, b1flat)

    logits = pl.pallas_call(
        _cls_body,
        grid=grid,
        in_specs=[_rows(NODE), _full((NODE, NODE // 2)), _full((1, NODE // 2)),
                  _full((NODE // 2, 1)), _full((1, 1))],
        out_specs=_rows(1),
        out_shape=jax.ShapeDtypeStruct((N, 1), jnp.float32),
    )(x2, cls_W1, cls_b1[None, :], cls_W2, cls_b2[None, :])

    return logits


# speedup vs baseline: 60.8003x; 60.8003x over previous
import functools

import jax
import jax.numpy as jnp
from jax import lax
from jax.experimental import pallas as pl
from jax.experimental.pallas import tpu as pltpu
from jax.experimental.pallas import tpu_sc as plsc

N = 10000
E = 320000
NODE = 128
EDGE = 16
TIME = 32
MEM = 128
HEADS = 4
HPD = 32
MSG = NODE + EDGE + TIME
L0_IN = NODE + MEM + TIME

BLK = 1000
NW = 32
EPW = E // NW
KW = 80
NWIN = EPW // KW
NPAD = 10240
NPS = NPAD // 16
WX = 144
SS0 = NODE
DN0 = NODE + HEADS


def _pack_ext(hh, s):
    b = hh.shape[0]
    return jnp.concatenate(
        [hh, s[:, :HEADS], jnp.ones((b, HEADS), jnp.float32),
         jnp.zeros((b, WX - DN0 - HEADS), jnp.float32)], axis=1)


def _pack_sd(s):
    b = s.shape[0]
    return jnp.concatenate(
        [s[:, HEADS:], jnp.zeros((b, 12), jnp.float32)], axis=1)


def _dense1_body(nf_ref, nfd2_ref, ea_ref, te_ref, has_ref,
                 w1a_ref, w1c_ref, w1d_ref, w1e_ref, b1_ref, w2_ref, b2_ref,
                 wiht_ref, bih_ref, bhh_ref, w0_ref, a0_ref,
                 hhx_ref, sd_ref):
    nf = nf_ref[...]
    u = (nf @ w1a_ref[...] + nfd2_ref[...] @ w1c_ref[...]
         + ea_ref[...] @ w1d_ref[...] + te_ref[...] @ w1e_ref[...] + b1_ref[...])
    u = jnp.maximum(u, 0.0)
    msgs = u @ w2_ref[...] + b2_ref[...]
    gi = msgs @ wiht_ref[...] + bih_ref[...]
    gh = bhh_ref[...]
    ir, iz, inn = gi[:, :MEM], gi[:, MEM:2 * MEM], gi[:, 2 * MEM:]
    hr, hz, hn = gh[:, :MEM], gh[:, MEM:2 * MEM], gh[:, 2 * MEM:]
    r = jax.nn.sigmoid(ir + hr)
    z = jax.nn.sigmoid(iz + hz)
    nn_ = jnp.tanh(inn + r * hn)
    newh = (1.0 - z) * nn_
    memv = newh * has_ref[...]
    x = jnp.concatenate([nf, memv, te_ref[...]], axis=1)
    hh0 = x @ w0_ref[...]
    s0 = hh0 @ a0_ref[...]
    hhx_ref[...] = _pack_ext(hh0, s0)
    sd_ref[...] = _pack_sd(s0)


def _unpack_finish(a0_ref, a1_ref, b_ref):
    acc = a0_ref[...] + a1_ref[...]
    parts = []
    for h in range(HEADS):
        parts.append(acc[:, h * HPD:(h + 1) * HPD]
                     / (acc[:, DN0 + h:DN0 + h + 1] + 1e-16))
    return jnp.concatenate(parts, axis=1) + b_ref[...]


def _finish_proj_body(a0_ref, a1_ref, b_ref, w_ref, am_ref, hhx_ref, sd_ref):
    x = _unpack_finish(a0_ref, a1_ref, b_ref)
    hh = x @ w_ref[...]
    s = hh @ am_ref[...]
    hhx_ref[...] = _pack_ext(hh, s)
    sd_ref[...] = _pack_sd(s)


def _finish_cls_body(a0_ref, a1_ref, b_ref, w1_ref, b1_ref, w2_ref, b2_ref,
                     out_ref):
    x = _unpack_finish(a0_ref, a1_ref, b_ref)
    u = jnp.maximum(x @ w1_ref[...] + b1_ref[...], 0.0)
    out_ref[...] = u @ w2_ref[...] + b2_ref[...]


def _full(shape):
    return pl.BlockSpec(shape, lambda i: (0,) * len(shape))


def _rows(width):
    return pl.BlockSpec((BLK, width), lambda i: (i, 0))


def _bcast_lane(vec, j):
    return lax.gather(
        vec, jnp.full((16, 1), j, jnp.int32),
        lax.GatherDimensionNumbers(offset_dims=(), collapsed_slice_dims=(0,),
                                   start_index_map=(0,)),
        (1,), mode=lax.GatherScatterMode.PROMISE_IN_BOUNDS)


def _shift4(vec):
    idx = jnp.maximum(lax.iota(jnp.int32, 16) - 4, 0)
    return lax.gather(
        vec, idx[:, None],
        lax.GatherDimensionNumbers(offset_dims=(), collapsed_slice_dims=(0,),
                                   start_index_map=(0,)),
        (1,), mode=lax.GatherScatterMode.PROMISE_IN_BOUNDS)


def _gat_sc_body(src_ref, dst_ref, hhx_ref, sd_ref, z_ref,
                 acc_ref,
                 idx_s, idx_d, rows, sdb, out_sp, sem):
    cid = lax.axis_index("c")
    sid = lax.axis_index("s")
    wid = sid * 2 + cid

    pltpu.sync_copy(z_ref.at[pl.ds(sid * NPS, NPS)],
                    out_sp.at[pl.ds(sid * NPS, NPS)])
    plsc.subcore_barrier()

    base_edge = wid * EPW

    def window(w, carry):
        wb = base_edge + w * KW
        pltpu.sync_copy(src_ref.at[pl.ds(wb, KW)], idx_s)
        pltpu.sync_copy(dst_ref.at[pl.ds(wb, KW)], idx_d)
        cp1 = pltpu.async_copy(hhx_ref.at[idx_s], rows, sem)
        cp2 = pltpu.async_copy(sd_ref.at[idx_d], sdb, sem)
        cp1.wait()
        cp2.wait()
        for j in range(KW):
            sseg = rows[j, pl.ds(SS0, 16)]
            v = sseg + sdb[j, :]
            v = jnp.where(v >= 0.0, v, 0.2 * v)
            w16 = jnp.exp(v)
            rows[j, pl.ds(SS0, 16)] = sseg * _shift4(w16)
            for h in range(HEADS):
                wh = _bcast_lane(w16, h)
                c0 = h * HPD
                rows[j, pl.ds(c0, 16)] = rows[j, pl.ds(c0, 16)] * wh
                rows[j, pl.ds(c0 + 16, 16)] = rows[j, pl.ds(c0 + 16, 16)] * wh
        pltpu.sync_copy(rows, out_sp.at[idx_d], add=True)
        return carry

    lax.fori_loop(0, NWIN, window, 0)
    plsc.subcore_barrier()
    pltpu.sync_copy(out_sp.at[pl.ds(sid * NPS, NPS)],
                    acc_ref.at[cid, pl.ds(sid * NPS, NPS)])


def _gat_aggregate(src, dst, hhx, sd, z):
    mesh = plsc.VectorSubcoreMesh(core_axis_name="c", subcore_axis_name="s")
    f = pl.kernel(
        _gat_sc_body,
        mesh=mesh,
        compiler_params=pltpu.CompilerParams(use_tc_tiling_on_sc=False,
                                             needs_layout_passes=False),
        out_type=jax.ShapeDtypeStruct((2, NPAD, WX), jnp.float32),
        scratch_types=[
            pltpu.VMEM((KW,), jnp.int32),
            pltpu.VMEM((KW,), jnp.int32),
            pltpu.VMEM((KW, WX), jnp.float32),
            pltpu.VMEM((KW, 16), jnp.float32),
            pltpu.VMEM_SHARED((NPAD, WX), jnp.float32),
            pltpu.SemaphoreType.DMA,
        ],
    )
    return f(src, dst, hhx, sd, z)


def kernel(node_features, edge_index, edge_attr, edge_times, time_w, time_b,
           msg_W1, msg_b1, msg_W2, msg_b2, gru_Wih, gru_Whh, gru_bih, gru_bhh,
           gat0_W, gat0_asrc, gat0_adst, gat0_b,
           gat1_W, gat1_asrc, gat1_adst, gat1_b,
           cls_W1, cls_b1, cls_W2, cls_b2):
    src = edge_index[0]
    dst = edge_index[1]
    eid = jnp.arange(E, dtype=jnp.int32)
    last_e = jnp.full((N,), -1, jnp.int32).at[src].max(eid)
    has = last_e >= 0
    le = jnp.maximum(last_e, 0)
    d2 = dst[le]
    nfd2 = node_features[d2]
    ea = edge_attr[le]
    t = jnp.where(has, edge_times[le], 0.0)
    te = jnp.sin(time_w * t[:, None] + time_b)
    has_f = has.astype(jnp.float32)[:, None]

    w1a = msg_W1[:NODE]
    w1c = msg_W1[NODE + MEM:NODE + MEM + NODE]
    w1d = msg_W1[NODE + MEM + NODE:NODE + MEM + NODE + EDGE]
    w1e = msg_W1[NODE + MEM + NODE + EDGE:]
    wiht = gru_Wih.T
    w0cat = jnp.transpose(gat0_W, (1, 0, 2)).reshape(L0_IN, HEADS * HPD)
    w1cat = jnp.transpose(gat1_W, (1, 0, 2)).reshape(NODE, HEADS * HPD)

    def amat(asrc, adst):
        a = jnp.zeros((HEADS * HPD, 2 * HEADS), jnp.float32)
        for h in range(HEADS):
            a = a.at[h * HPD:(h + 1) * HPD, h].set(asrc[h])
            a = a.at[h * HPD:(h + 1) * HPD, HEADS + h].set(adst[h])
        return a

    a0 = amat(gat0_asrc, gat0_adst)
    a1 = amat(gat1_asrc, gat1_adst)
    b0flat = gat0_b.reshape(1, -1)
    b1flat = gat1_b.reshape(1, -1)
    z = jnp.zeros((NPAD, WX), jnp.float32)

    grid = (N // BLK,)
    hhx0, sd0 = pl.pallas_call(
        _dense1_body,
        grid=grid,
        in_specs=[
            _rows(NODE), _rows(NODE), _rows(EDGE), _rows(TIME), _rows(1),
            _full((NODE, MSG)), _full((NODE, MSG)), _full((EDGE, MSG)),
            _full((TIME, MSG)), _full((1, MSG)), _full((MSG, MSG)),
            _full((1, MSG)), _full((MSG, 3 * MEM)), _full((1, 3 * MEM)),
            _full((1, 3 * MEM)), _full((L0_IN, HEADS * HPD)),
            _full((HEADS * HPD, 2 * HEADS)),
        ],
        out_specs=[_rows(WX), _rows(16)],
        out_shape=[
            jax.ShapeDtypeStruct((N, WX), jnp.float32),
            jax.ShapeDtypeStruct((N, 16), jnp.float32),
        ],
    )(node_features, nfd2, ea, te, has_f, w1a, w1c, w1d, w1e,
      msg_b1[None, :], msg_W2, msg_b2[None, :],
      wiht, gru_bih[None, :], gru_bhh[None, :], w0cat, a0)

    acc0 = _gat_aggregate(src, dst, hhx0, sd0, z)

    hhx1, sd1 = pl.pallas_call(
        _finish_proj_body,
        grid=grid,
        in_specs=[_rows(WX), _rows(WX), _full((1, HEADS * HPD)),
                  _full((NODE, HEADS * HPD)), _full((HEADS * HPD, 2 * HEADS))],
        out_specs=[_rows(WX), _rows(16)],
        out_shape=[
            jax.ShapeDtypeStruct((N, WX), jnp.float32),
            jax.ShapeDtypeStruct((N, 16), jnp.float32),
        ],
    )(acc0[0], acc0[1], b0flat, w1cat, a1)

    acc1 = _gat_aggregate(src, dst, hhx1, sd1, z)

    logits = pl.pallas_call(
        _finish_cls_body,
        grid=grid,
        in_specs=[_rows(WX), _rows(WX), _full((1, HEADS * HPD)),
                  _full((NODE, NODE // 2)), _full((1, NODE // 2)),
                  _full((NODE // 2, 1)), _full((1, 1))],
        out_specs=_rows(1),
        out_shape=jax.ShapeDtypeStruct((N, 1), jnp.float32),
    )(acc1[0], acc1[1], b1flat,
      cls_W1, cls_b1[None, :], cls_W2, cls_b2[None, :])

    return logits
